# baseline (speedup 1.0000x reference)
import functools

import jax
import jax.numpy as jnp
from jax import lax
from jax.experimental import pallas as pl
from jax.experimental.pallas import tpu as pltpu
from jax.experimental.pallas import tpu_sc as plsc

B = 16384
F = 26
V = 100000
D = 32
ND = 16

NC = 2
NS = 16
NW = NC * NS
RPT = B // NW
CH = 32
NCHUNK = RPT // CH
IPC = CH * F
SUB = 104
NSUB = IPC // SUB


def _fm2_body(idx_hbm, xd_hbm, e1_hbm, e2_hbm, wd_hbm, bd_hbm, out_hbm,
              idxb0, idxb1, gb0, gb1, e1b0, e1b1, xdb, wdb, bdb, outb,
              sg0, sg1, se0, se1):
    wid = lax.axis_index("s") * NC + lax.axis_index("c")
    rows0 = wid * RPT
    ibase = rows0 * F

    pltpu.sync_copy(xd_hbm.at[pl.ds(rows0, RPT)], xdb)
    pltpu.sync_copy(wd_hbm, wdb)
    pltpu.sync_copy(bd_hbm, bdb)

    wv = wdb[...]
    bdv = bdb[...]
    lanes = lax.iota(jnp.int32, 16)
    lanesF = lanes * F
    zero = jnp.zeros((16,), jnp.float32)

    gdn = lax.GatherDimensionNumbers(
        offset_dims=(), collapsed_slice_dims=(0,), start_index_map=(0,))

    def lane_shuffle(x, idx):
        return lax.gather(x, idx[:, None], dimension_numbers=gdn,
                          slice_sizes=(1,),
                          mode=lax.GatherScatterMode.PROMISE_IN_BOUNDS)

    def lane_allsum(x):
        for s in (1, 2, 4, 8):
            x = x + lane_shuffle(x, lanes ^ s)
        return x

    bufs = ((idxb0, gb0, e1b0, sg0, se0), (idxb1, gb1, e1b1, sg1, se1))

    def stage(c, par):
        idxb, gbuf, e1b, sg, se = bufs[par]
        pltpu.sync_copy(idx_hbm.at[pl.ds(ibase + c * IPC, IPC)], idxb)
        for j in range(NSUB):
            sl = pl.ds(j * SUB, SUB)
            pltpu.async_copy(e2_hbm.at[idxb.at[sl]], gbuf.at[sl], sg)
            pltpu.async_copy(e1_hbm.at[idxb.at[sl]], e1b.at[sl], se)

    def drain(par):
        idxb, gbuf, e1b, sg, se = bufs[par]
        pltpu.make_async_copy(e2_hbm.at[pl.ds(0, IPC)], gbuf, sg).wait()
        pltpu.make_async_copy(e1_hbm.at[pl.ds(0, IPC)], e1b, se).wait()

    def compute(c, par):
        _, gbuf, e1b, _, _ = bufs[par]

        for grp in range(CH // 16):
            gb = grp * 16

            def row_body(i, rpack):
                base = (gb + i) * F

                def f_body(f, acc):
                    s0, s1, sq = acc
                    a = gbuf[base + f, pl.ds(0, 16)]
                    b = gbuf[base + f, pl.ds(16, 16)]
                    return (s0 + a, s1 + b, sq + a * a + b * b)

                s0, s1, sq = lax.fori_loop(0, F, f_body, (zero, zero, zero))
                xv = xdb[c * CH + gb + i, pl.ds(0, 16)]
                rvec = 0.5 * (s0 * s0 + s1 * s1 - sq) + xv * wv
                return jnp.where(lanes == i, lane_allsum(rvec), rpack)

            rpack = lax.fori_loop(0, 16, row_body, zero)

            def e1_body(f, acc):
                return acc + plsc.load_gather(e1b, [lanesF + (gb * F + f)])

            s1sum = lax.fori_loop(0, F, e1_body, zero)
            v = rpack + s1sum + bdv
            outb[pl.ds(c * CH + gb, 16)] = 1.0 / (1.0 + jnp.exp(-v))

    stage(0, 0)
    for c in range(NCHUNK):
        par = c % 2
        if c + 1 < NCHUNK:
            stage(c + 1, 1 - par)
        drain(par)
        compute(c, par)

    pltpu.sync_copy(outb, out_hbm.at[pl.ds(rows0, RPT)])


@functools.partial(
    pl.kernel,
    out_type=jax.ShapeDtypeStruct((B,), jnp.float32),
    mesh=plsc.VectorSubcoreMesh(core_axis_name="c", subcore_axis_name="s"),
    compiler_params=pltpu.CompilerParams(needs_layout_passes=False,
                                         use_tc_tiling_on_sc=False),
    scratch_types=[
        pltpu.VMEM((IPC,), jnp.int32),
        pltpu.VMEM((IPC,), jnp.int32),
        pltpu.VMEM((IPC, D), jnp.float32),
        pltpu.VMEM((IPC, D), jnp.float32),
        pltpu.VMEM((IPC,), jnp.float32),
        pltpu.VMEM((IPC,), jnp.float32),
        pltpu.VMEM((RPT, ND), jnp.float32),
        pltpu.VMEM((16,), jnp.float32),
        pltpu.VMEM((16,), jnp.float32),
        pltpu.VMEM((RPT,), jnp.float32),
        pltpu.SemaphoreType.DMA,
        pltpu.SemaphoreType.DMA,
        pltpu.SemaphoreType.DMA,
        pltpu.SemaphoreType.DMA,
    ],
)
def _fm2_sc(idx_hbm, xd_hbm, e1_hbm, e2_hbm, wd_hbm, bd_hbm, out_hbm, *rest):
    _fm2_body(idx_hbm, xd_hbm, e1_hbm, e2_hbm, wd_hbm, bd_hbm, out_hbm, *rest)


def kernel(X_sparse, X_dense, emb1, emb2, Wd, bd):
    idx_flat = (X_sparse.astype(jnp.int32)
                + jnp.arange(F, dtype=jnp.int32)[None, :] * V).reshape(-1)
    e1_flat = emb1.reshape(F * V)
    e2_flat = emb2.reshape(F * V, D)
    wd_flat = Wd.reshape(ND)
    bd16 = jnp.broadcast_to(bd, (16,))
    out = _fm2_sc(idx_flat, X_dense, e1_flat, e2_flat, wd_flat, bd16)
    return out.reshape(B, 1)
